# baseline (speedup 1.0000x reference)
import functools

import jax
import jax.numpy as jnp
from jax import lax
from jax.experimental import pallas as pl
from jax.experimental.pallas import tpu as pltpu
from jax.experimental.pallas import tpu_sc as plsc

_N = 10000
_F = 128
_FH = 64
_NG = 64
_NC = 10
_E = 320000
_CH = 64
_NCHUNK = 160
_CHA = 128
_NCH0 = 152
_NCH1 = 8
_NPH = 19
_EPAD = 32 * _NCHUNK * _CH
_GBYTES = _CH * _F * 4
_STRIPE = 632
_NPAD = 16 * _STRIPE

_f32 = jnp.float32

_mesh = plsc.VectorSubcoreMesh(core_axis_name="c", subcore_axis_name="s",
                               num_cores=2, num_subcores=16)


@functools.partial(
    pl.kernel,
    out_type=jax.ShapeDtypeStruct((2, _NPAD, _F), _f32),
    mesh=_mesh,
    scratch_types=[
        pltpu.VMEM((_NCHUNK, _CH), jnp.int32),
        pltpu.VMEM((_CH, _F), _f32),
        pltpu.VMEM_SHARED((_NPAD, _F), _f32),
    ],
)
def _sc_deg(dst_hbm, ones_hbm, zeros_hbm, out_hbm, didx, onesb, acc):
    c = lax.axis_index("c")
    s = lax.axis_index("s")
    wid = c * 16 + s
    pltpu.sync_copy(dst_hbm.at[wid], didx)
    pltpu.sync_copy(ones_hbm, onesb)
    pltpu.sync_copy(zeros_hbm.at[pl.ds(s * _STRIPE, _STRIPE)],
                    acc.at[pl.ds(s * _STRIPE, _STRIPE)])
    plsc.subcore_barrier()

    def body(j, carry):
        pltpu.sync_copy(onesb, acc.at[didx.at[j]], add=True)
        return carry

    lax.fori_loop(0, _NCHUNK, body, 0)
    plsc.subcore_barrier()
    pltpu.sync_copy(acc.at[pl.ds(s * _STRIPE, _STRIPE)],
                    out_hbm.at[c].at[pl.ds(s * _STRIPE, _STRIPE)])


@functools.partial(
    pl.kernel,
    out_type=jax.ShapeDtypeStruct((2, _NPAD, _F), _f32),
    mesh=_mesh,
    scratch_types=[
        pltpu.VMEM((8, _CHA), jnp.int32),
        pltpu.VMEM((8, _CHA), jnp.int32),
        pltpu.VMEM((_CHA, _F), _f32),
        pltpu.VMEM((_CHA, _F), _f32),
        pltpu.VMEM_SHARED((_NPAD, _F), _f32),
        pltpu.SemaphoreType.DMA,
    ],
)
def _sc_agg(g_hbm, src_hbm, dst_hbm, zeros_hbm, out_hbm,
            sidx, didx, rows0, rows1, acc, semA):
    c = lax.axis_index("c")
    s = lax.axis_index("s")
    pltpu.sync_copy(zeros_hbm.at[pl.ds(s * _STRIPE, _STRIPE)],
                    acc.at[pl.ds(s * _STRIPE, _STRIPE)])
    plsc.subcore_barrier()

    def dbl_loop(n):
        def body(jj, carry):
            j0 = 2 * jj
            pltpu.async_copy(g_hbm.at[sidx.at[j0 + 1]], rows1, semA)
            pltpu.make_async_copy(g_hbm.at[pl.ds(0, _CHA)],
                                  rows0, semA).wait()
            pltpu.sync_copy(rows0, acc.at[didx.at[j0]], add=True)

            @pl.when(jj < n - 1)
            def _():
                pltpu.async_copy(g_hbm.at[sidx.at[j0 + 2]], rows0, semA)

            pltpu.make_async_copy(g_hbm.at[pl.ds(0, _CHA)],
                                  rows1, semA).wait()
            pltpu.sync_copy(rows1, acc.at[didx.at[j0 + 1]], add=True)
            return carry

        lax.fori_loop(0, n, body, 0)

    def phase(h, pcarry):
        @pl.when(c == 0)
        def _():
            row0 = pl.multiple_of(s * _NCH0 + h * 8, 8)
            pltpu.sync_copy(src_hbm.at[pl.ds(row0, 8)], sidx)
            pltpu.sync_copy(dst_hbm.at[pl.ds(row0, 8)], didx)
            pltpu.async_copy(g_hbm.at[sidx.at[0]], rows0, semA)
            dbl_loop(4)

        @pl.when((c == 1) & (h < 1))
        def _():
            row1 = pl.multiple_of(16 * _NCH0 + s * _NCH1 + h * 8, 8)
            pltpu.sync_copy(src_hbm.at[pl.ds(row1, 8)],
                            sidx.at[pl.ds(0, 8)])
            pltpu.sync_copy(dst_hbm.at[pl.ds(row1, 8)],
                            didx.at[pl.ds(0, 8)])
            pltpu.async_copy(g_hbm.at[sidx.at[0]], rows0, semA)
            dbl_loop(4)
        return pcarry

    lax.fori_loop(0, _NPH, phase, 0)
    plsc.subcore_barrier()
    pltpu.sync_copy(acc.at[pl.ds(s * _STRIPE, _STRIPE)],
                    out_hbm.at[c].at[pl.ds(s * _STRIPE, _STRIPE)])


def _tc_first(x_ref, w_ref, degp_ref, g_ref, dinv_ref):
    deg = 1.0 + degp_ref[0, :, :1] + degp_ref[1, :, :1]
    dinv = lax.rsqrt(deg)
    dinv_ref[...] = jnp.broadcast_to(dinv, (_NPAD, _F))
    h = jnp.dot(x_ref[...], w_ref[...], preferred_element_type=jnp.float32)
    g_ref[...] = h * dinv_ref[: _N, :]


def _tc_mid(agg_ref, g_ref, dinv_ref, b_ref, w_ref, o_ref):
    agg = agg_ref[0, : _N, :] + agg_ref[1, : _N, :] + g_ref[...]
    d = dinv_ref[: _N, :]
    a = jnp.maximum(d * agg + b_ref[...], 0.0)
    o_ref[...] = jnp.dot(a, w_ref[...],
                         preferred_element_type=jnp.float32) * d


def _tc_pool(agg_ref, g_ref, dinv_ref, b_ref, batch_ref, wl_ref, bl_ref,
             o_ref):
    agg = agg_ref[0, : _N, :] + agg_ref[1, : _N, :] + g_ref[...]
    d = dinv_ref[: _N, :]
    a = jnp.maximum(d * agg + b_ref[...], 0.0)
    gid = lax.broadcasted_iota(jnp.int32, (_NG, _N), 0)
    m = (batch_ref[...] == gid).astype(jnp.float32)
    sums = jnp.dot(m, a, preferred_element_type=jnp.float32)
    cnt = jnp.dot(m, jnp.ones((_N, 1), jnp.float32),
                  preferred_element_type=jnp.float32)
    pooled = sums / jnp.maximum(cnt, 1.0)
    logits = jnp.dot(pooled, wl_ref[...],
                     preferred_element_type=jnp.float32) + bl_ref[...]
    mx = jnp.max(logits, axis=1, keepdims=True)
    e = jnp.exp(logits - mx)
    lse = jnp.log(jnp.sum(e, axis=1, keepdims=True))
    o_ref[...] = logits - mx - lse


_tc_first_call = pl.pallas_call(
    _tc_first,
    out_shape=(jax.ShapeDtypeStruct((_N, _F), _f32),
               jax.ShapeDtypeStruct((_NPAD, _F), _f32)),
)

_tc_mid_call = pl.pallas_call(
    _tc_mid,
    out_shape=jax.ShapeDtypeStruct((_N, _F), _f32),
)

_tc_pool_call = pl.pallas_call(
    _tc_pool,
    out_shape=jax.ShapeDtypeStruct((_NG, _NC), _f32),
)


def kernel(x, edge_index, batch, W1, b1, W2, b2, W3, b3, Wl, bl):
    src = edge_index[0].astype(jnp.int32)
    dst = edge_index[1].astype(jnp.int32)
    pad = _EPAD - _E
    srcf = jnp.concatenate([src, jnp.zeros((pad,), jnp.int32)])
    dstf = jnp.concatenate([dst, jnp.full((pad,), _NPAD - 1, jnp.int32)])
    srcp = srcf.reshape(_EPAD // _CHA, _CHA)
    dstp = dstf.reshape(_EPAD // _CHA, _CHA)
    dstp_deg = dstf.reshape(32, _NCHUNK, _CH)

    ones_blk = jnp.ones((_CH, _F), _f32)
    zeros_full = jnp.zeros((_NPAD, _F), _f32)

    degp = _sc_deg(dstp_deg, ones_blk, zeros_full)
    g1, dinv = _tc_first_call(x, W1, degp)
    agg1 = _sc_agg(g1, srcp, dstp, zeros_full)
    g2 = _tc_mid_call(agg1, g1, dinv, b1.reshape(1, _F), W2)
    agg2 = _sc_agg(g2, srcp, dstp, zeros_full)
    g3 = _tc_mid_call(agg2, g2, dinv, b2.reshape(1, _F), W3)
    agg3 = _sc_agg(g3, srcp, dstp, zeros_full)
    out = _tc_pool_call(agg3, g3, dinv, b3.reshape(1, _F),
                        batch.reshape(1, _N).astype(jnp.int32),
                        Wl, bl.reshape(1, _NC))
    return out


# speedup vs baseline: 1.0004x; 1.0004x over previous
import functools

import jax
import jax.numpy as jnp
from jax import lax
from jax.experimental import pallas as pl
from jax.experimental.pallas import tpu as pltpu
from jax.experimental.pallas import tpu_sc as plsc

_N = 10000
_F = 128
_NG = 64
_NC = 10
_E = 320000
_CH = 64
_NCHUNK = 160
_CHA = 128
_NCH0 = 144
_NCH1 = 16
_NPH = 9
_EPAD = 32 * _NCHUNK * _CH
_STRIPE = 632
_NPAD = 16 * _STRIPE

_f32 = jnp.float32

_mesh = plsc.VectorSubcoreMesh(core_axis_name="c", subcore_axis_name="s",
                               num_cores=2, num_subcores=16)


@functools.partial(
    pl.kernel,
    out_type=jax.ShapeDtypeStruct((2, _NPAD, _F), _f32),
    mesh=_mesh,
    scratch_types=[
        pltpu.VMEM((_NCHUNK, _CH), jnp.int32),
        pltpu.VMEM((_CH, _F), _f32),
        pltpu.VMEM_SHARED((_NPAD, _F), _f32),
    ],
)
def _sc_deg(dst_hbm, ones_hbm, zeros_hbm, out_hbm, didx, onesb, acc):
    c = lax.axis_index("c")
    s = lax.axis_index("s")
    wid = c * 16 + s
    pltpu.sync_copy(dst_hbm.at[wid], didx)
    pltpu.sync_copy(ones_hbm, onesb)
    pltpu.sync_copy(zeros_hbm.at[pl.ds(s * _STRIPE, _STRIPE)],
                    acc.at[pl.ds(s * _STRIPE, _STRIPE)])
    plsc.subcore_barrier()

    def body(j, carry):
        pltpu.sync_copy(onesb, acc.at[didx.at[j]], add=True)
        return carry

    lax.fori_loop(0, _NCHUNK, body, 0)
    plsc.subcore_barrier()
    pltpu.sync_copy(acc.at[pl.ds(s * _STRIPE, _STRIPE)],
                    out_hbm.at[c].at[pl.ds(s * _STRIPE, _STRIPE)])


@functools.partial(
    pl.kernel,
    out_type=jax.ShapeDtypeStruct((2, _NPAD, _F), _f32),
    mesh=_mesh,
    scratch_types=[
        pltpu.VMEM((16, _CHA), jnp.int32),
        pltpu.VMEM((16, _CHA), jnp.int32),
        pltpu.VMEM((_CHA, _F), _f32),
        pltpu.VMEM((_CHA, _F), _f32),
        pltpu.VMEM_SHARED((_NPAD, _F), _f32),
        pltpu.SemaphoreType.DMA,
    ],
)
def _sc_agg(g_hbm, src_hbm, dst_hbm, zeros_hbm, out_hbm,
            sidx, didx, rows0, rows1, acc, semA):
    c = lax.axis_index("c")
    s = lax.axis_index("s")
    pltpu.sync_copy(zeros_hbm.at[pl.ds(s * _STRIPE, _STRIPE)],
                    acc.at[pl.ds(s * _STRIPE, _STRIPE)])
    plsc.subcore_barrier()

    def dbl_loop(n):
        def body(jj, carry):
            j0 = 2 * jj
            pltpu.async_copy(g_hbm.at[sidx.at[j0 + 1]], rows1, semA)
            pltpu.make_async_copy(g_hbm.at[pl.ds(0, _CHA)],
                                  rows0, semA).wait()
            pltpu.sync_copy(rows0, acc.at[didx.at[j0]], add=True)

            @pl.when(jj < n - 1)
            def _():
                pltpu.async_copy(g_hbm.at[sidx.at[j0 + 2]], rows0, semA)

            pltpu.make_async_copy(g_hbm.at[pl.ds(0, _CHA)],
                                  rows1, semA).wait()
            pltpu.sync_copy(rows1, acc.at[didx.at[j0 + 1]], add=True)
            return carry

        lax.fori_loop(0, n, body, 0)

    def phase(h, pcarry):
        @pl.when(c == 0)
        def _():
            row0 = pl.multiple_of(s * _NCH0 + h * 16, 8)
            pltpu.sync_copy(src_hbm.at[pl.ds(row0, 16)], sidx)
            pltpu.sync_copy(dst_hbm.at[pl.ds(row0, 16)], didx)
            pltpu.async_copy(g_hbm.at[sidx.at[0]], rows0, semA)
            dbl_loop(8)

        @pl.when((c == 1) & (h < 2))
        def _():
            row1 = pl.multiple_of(16 * _NCH0 + s * _NCH1 + h * 8, 8)
            pltpu.sync_copy(src_hbm.at[pl.ds(row1, 8)],
                            sidx.at[pl.ds(0, 8)])
            pltpu.sync_copy(dst_hbm.at[pl.ds(row1, 8)],
                            didx.at[pl.ds(0, 8)])
            pltpu.async_copy(g_hbm.at[sidx.at[0]], rows0, semA)
            dbl_loop(4)
        return pcarry

    lax.fori_loop(0, _NPH, phase, 0)
    plsc.subcore_barrier()
    pltpu.sync_copy(acc.at[pl.ds(s * _STRIPE, _STRIPE)],
                    out_hbm.at[c].at[pl.ds(s * _STRIPE, _STRIPE)])


def _tc_first(x_ref, w_ref, degp_ref, g_ref, dinv_ref):
    deg = 1.0 + degp_ref[0, :, :1] + degp_ref[1, :, :1]
    dinv = lax.rsqrt(deg)
    dinv_ref[...] = jnp.broadcast_to(dinv, (_NPAD, _F))
    h = jnp.dot(x_ref[...], w_ref[...], preferred_element_type=jnp.float32)
    g_ref[...] = h * dinv_ref[: _N, :]


def _tc_mid(agg_ref, g_ref, dinv_ref, b_ref, w_ref, o_ref):
    agg = agg_ref[0, : _N, :] + agg_ref[1, : _N, :] + g_ref[...]
    d = dinv_ref[: _N, :]
    a = jnp.maximum(d * agg + b_ref[...], 0.0)
    o_ref[...] = jnp.dot(a, w_ref[...],
                         preferred_element_type=jnp.float32) * d


def _tc_pool(agg_ref, g_ref, dinv_ref, b_ref, batch_ref, wl_ref, bl_ref,
             o_ref):
    agg = agg_ref[0, : _N, :] + agg_ref[1, : _N, :] + g_ref[...]
    d = dinv_ref[: _N, :]
    a = jnp.maximum(d * agg + b_ref[...], 0.0)
    gid = lax.broadcasted_iota(jnp.int32, (_NG, _N), 0)
    m = (batch_ref[...] == gid).astype(jnp.float32)
    sums = jnp.dot(m, a, preferred_element_type=jnp.float32)
    cnt = jnp.dot(m, jnp.ones((_N, 1), jnp.float32),
                  preferred_element_type=jnp.float32)
    pooled = sums / jnp.maximum(cnt, 1.0)
    logits = jnp.dot(pooled, wl_ref[...],
                     preferred_element_type=jnp.float32) + bl_ref[...]
    mx = jnp.max(logits, axis=1, keepdims=True)
    e = jnp.exp(logits - mx)
    lse = jnp.log(jnp.sum(e, axis=1, keepdims=True))
    o_ref[...] = logits - mx - lse


_tc_first_call = pl.pallas_call(
    _tc_first,
    out_shape=(jax.ShapeDtypeStruct((_N, _F), _f32),
               jax.ShapeDtypeStruct((_NPAD, _F), _f32)),
)

_tc_mid_call = pl.pallas_call(
    _tc_mid,
    out_shape=jax.ShapeDtypeStruct((_N, _F), _f32),
)

_tc_pool_call = pl.pallas_call(
    _tc_pool,
    out_shape=jax.ShapeDtypeStruct((_NG, _NC), _f32),
)


def kernel(x, edge_index, batch, W1, b1, W2, b2, W3, b3, Wl, bl):
    src = edge_index[0].astype(jnp.int32)
    dst = edge_index[1].astype(jnp.int32)
    pad = _EPAD - _E
    srcf = jnp.concatenate([src, jnp.zeros((pad,), jnp.int32)])
    dstf = jnp.concatenate([dst, jnp.full((pad,), _NPAD - 1, jnp.int32)])
    srcp = srcf.reshape(_EPAD // _CHA, _CHA)
    dstp = dstf.reshape(_EPAD // _CHA, _CHA)
    dstp_deg = dstf.reshape(32, _NCHUNK, _CH)

    ones_blk = jnp.ones((_CH, _F), _f32)
    zeros_full = jnp.zeros((_NPAD, _F), _f32)

    degp = _sc_deg(dstp_deg, ones_blk, zeros_full)
    g1, dinv = _tc_first_call(x, W1, degp)
    agg1 = _sc_agg(g1, srcp, dstp, zeros_full)
    g2 = _tc_mid_call(agg1, g1, dinv, b1.reshape(1, _F), W2)
    agg2 = _sc_agg(g2, srcp, dstp, zeros_full)
    g3 = _tc_mid_call(agg2, g2, dinv, b2.reshape(1, _F), W3)
    agg3 = _sc_agg(g3, srcp, dstp, zeros_full)
    out = _tc_pool_call(agg3, g3, dinv, b3.reshape(1, _F),
                        batch.reshape(1, _N).astype(jnp.int32),
                        Wl, bl.reshape(1, _NC))
    return out
